# baseline (device time: 9614 ns/iter reference)
import jax
import jax.numpy as jnp
from jax import lax
from jax.experimental import pallas as pl
from jax.experimental.pallas import tpu as pltpu


def kernel(x, dest):
    m, n = x.shape
    g = 2 * m

    def body(x_ref, dest_ref, out_ref, xv_ref, d1_ref, xbf_ref, pxbf_ref,
             gd_ref, d2_ref, vout_ref, local_sems, send_sems, recv_sems):
        my_x = lax.axis_index("x")
        my_y = lax.axis_index("y")
        peer = (1 - my_x, my_y)

        with jax.named_scope("phase#p=load_start"):
            cd = pltpu.make_async_copy(dest_ref, d1_ref, local_sems.at[1])
            cd.start()
            cx = pltpu.make_async_copy(x_ref, xv_ref, local_sems.at[0])
            cx.start()

        with jax.named_scope("phase#p=barrier"):
            barrier_sem = pltpu.get_barrier_semaphore()
            pl.semaphore_signal(
                barrier_sem, inc=1, device_id=peer,
                device_id_type=pl.DeviceIdType.MESH,
            )
            pl.semaphore_wait(barrier_sem, 1)

        row0 = my_x * m
        with jax.named_scope("phase#p=send_dest"):
            cd.wait()
            d2_ref[...] = d1_ref[...].reshape(1, m)
            gd_ref[:, pl.ds(row0, m)] = d2_ref[...]
            rd = pltpu.make_async_remote_copy(
                src_ref=d2_ref,
                dst_ref=gd_ref.at[:, pl.ds(row0, m)],
                send_sem=send_sems.at[1],
                recv_sem=recv_sems.at[1],
                device_id=peer,
                device_id_type=pl.DeviceIdType.MESH,
            )
            rd.start()

        with jax.named_scope("phase#p=conv_send_x"):
            cx.wait()
            xbf_ref[...] = xv_ref[...].astype(jnp.bfloat16)
            rx = pltpu.make_async_remote_copy(
                src_ref=xbf_ref,
                dst_ref=pxbf_ref,
                send_sem=send_sems.at[0],
                recv_sem=recv_sems.at[0],
                device_id=peer,
                device_id_type=pl.DeviceIdType.MESH,
            )
            rx.start()

        with jax.named_scope("phase#p=wait_dest"):
            rd.wait()
        with jax.named_scope("phase#p=rank"):
            d = gd_ref[...]
            match = d == my_x
            mf = jnp.where(match, 1.0, 0.0)
            s = mf
            sh = 1
            while sh < g:
                s = s + jnp.concatenate(
                    [jnp.zeros((1, sh), jnp.float32), s[:, : g - sh]], axis=1
                )
                sh *= 2
            rankm = jnp.where(match, s - 1.0, -1.0).astype(jnp.int32)

            r_own = jnp.where(my_x == 0, rankm[:, :m], rankm[:, m:])
            r_peer = jnp.where(my_x == 0, rankm[:, m:], rankm[:, :m])

        with jax.named_scope("phase#p=p_own_mm"):
            j = lax.broadcasted_iota(jnp.int32, (m, m), 0)
            p_own = jnp.where(j == r_own, 1.0, 0.0).astype(jnp.bfloat16)
            acc = jnp.dot(
                p_own, xbf_ref[...], preferred_element_type=jnp.float32
            )
            p_peer = jnp.where(j == r_peer, 1.0, 0.0).astype(jnp.bfloat16)

        with jax.named_scope("phase#p=wait_x"):
            rx.wait()
        with jax.named_scope("phase#p=p_peer_mm"):
            vout_ref[...] = acc + jnp.dot(
                p_peer, pxbf_ref[...], preferred_element_type=jnp.float32
            )
        with jax.named_scope("phase#p=store_out"):
            co = pltpu.make_async_copy(vout_ref, out_ref, local_sems.at[2])
            co.start()
            co.wait()

    return pl.pallas_call(
        body,
        out_shape=jax.ShapeDtypeStruct((m, n), jnp.float32),
        in_specs=[
            pl.BlockSpec(memory_space=pl.ANY),
            pl.BlockSpec(memory_space=pl.ANY),
        ],
        out_specs=pl.BlockSpec(memory_space=pl.ANY),
        scratch_shapes=[
            pltpu.VMEM((m, n), jnp.float32),
            pltpu.VMEM((m,), jnp.int32),
            pltpu.VMEM((m, n), jnp.bfloat16),
            pltpu.VMEM((m, n), jnp.bfloat16),
            pltpu.VMEM((1, g), jnp.int32),
            pltpu.VMEM((1, m), jnp.int32),
            pltpu.VMEM((m, n), jnp.float32),
            pltpu.SemaphoreType.DMA((3,)),
            pltpu.SemaphoreType.DMA((2,)),
            pltpu.SemaphoreType.DMA((2,)),
        ],
        compiler_params=pltpu.CompilerParams(collective_id=0),
    )(x, dest)


# device time: 9261 ns/iter; 1.0381x vs baseline; 1.0381x over previous
import jax
import jax.numpy as jnp
from jax import lax
from jax.experimental import pallas as pl
from jax.experimental.pallas import tpu as pltpu

N_CH = 8
CH = 64


def kernel(x, dest):
    m, n = x.shape
    assert N_CH * CH == m

    def body(x_ref, dest_ref, out_ref, xbf_ref, sbuf_ref, pbuf_ref,
             d2_ref, pd_ref, send_sems, recv_sems):
        my_x = lax.axis_index("x")
        my_y = lax.axis_index("y")
        peer = (1 - my_x, my_y)

        pbuf_ref[...] = jnp.zeros((m, n), jnp.bfloat16)

        barrier_sem = pltpu.get_barrier_semaphore()
        pl.semaphore_signal(
            barrier_sem, inc=1, device_id=peer,
            device_id_type=pl.DeviceIdType.MESH,
        )
        pl.semaphore_wait(barrier_sem, 1)

        d2_ref[...] = dest_ref[...].reshape(1, m)
        rd = pltpu.make_async_remote_copy(
            src_ref=d2_ref,
            dst_ref=pd_ref,
            send_sem=send_sems.at[N_CH],
            recv_sem=recv_sems.at[N_CH],
            device_id=peer,
            device_id_type=pl.DeviceIdType.MESH,
        )
        rd.start()

        d2 = d2_ref[...]
        keep = d2 == my_x
        kf = jnp.where(keep, 1.0, 0.0)
        ksum = kf
        sh = 1
        while sh < m:
            ksum = ksum + jnp.concatenate(
                [jnp.zeros((1, sh), jnp.float32), ksum[:, : m - sh]], axis=1
            )
            sh *= 2
        ksum_i = ksum.astype(jnp.int32)
        lane = lax.broadcasted_iota(jnp.int32, (1, m), 1)
        srank = lane - ksum_i
        c_keep = jnp.sum(kf).astype(jnp.int32)
        c_send = m - c_keep

        xbf_ref[...] = x_ref[...].astype(jnp.bfloat16)
        spos = jnp.where(keep, -1, srank)
        jt = lax.broadcasted_iota(jnp.int32, (m, m), 0)
        p_send = jnp.where(jt == spos, 1.0, 0.0).astype(jnp.bfloat16)
        sbuf_ref[...] = jnp.dot(
            p_send, xbf_ref[...], preferred_element_type=jnp.float32
        ).astype(jnp.bfloat16)

        n_send = (c_send + CH - 1) // CH
        descs = []
        for i in range(N_CH):
            desc = pltpu.make_async_remote_copy(
                src_ref=sbuf_ref.at[pl.ds(i * CH, CH), :],
                dst_ref=pbuf_ref.at[pl.ds(i * CH, CH), :],
                send_sem=send_sems.at[i],
                recv_sem=recv_sems.at[i],
                device_id=peer,
                device_id_type=pl.DeviceIdType.MESH,
            )
            descs.append(desc)

            @pl.when(i < n_send)
            def _(desc=desc):
                desc.start()

        rd.wait()
        pd = pd_ref[...]
        c_recv = jnp.sum(jnp.where(pd == my_x, 1, 0)).astype(jnp.int32)
        own_off = jnp.where(my_x == 0, 0, c_recv)
        peer_off = jnp.where(my_x == 0, c_keep, 0)

        own_pos = jnp.where(keep, ksum_i - 1 + own_off, -1)
        p_own = jnp.where(jt == own_pos, 1.0, 0.0).astype(jnp.bfloat16)
        acc = jnp.dot(p_own, xbf_ref[...], preferred_element_type=jnp.float32)

        tt = lax.broadcasted_iota(jnp.int32, (m, m), 1)
        p_peer = jnp.where(
            (jt - tt == peer_off) & (tt < c_recv), 1.0, 0.0
        ).astype(jnp.bfloat16)

        n_recv = (c_recv + CH - 1) // CH
        for i in range(N_CH):
            @pl.when(i < n_recv)
            def _(desc=descs[i]):
                desc.wait_recv()

        out_ref[...] = acc + jnp.dot(
            p_peer, pbuf_ref[...], preferred_element_type=jnp.float32
        )

        for i in range(N_CH):
            @pl.when(i < n_send)
            def _(desc=descs[i]):
                desc.wait_send()

    return pl.pallas_call(
        body,
        out_shape=jax.ShapeDtypeStruct((m, n), jnp.float32),
        in_specs=[
            pl.BlockSpec(memory_space=pltpu.VMEM),
            pl.BlockSpec(memory_space=pltpu.VMEM),
        ],
        out_specs=pl.BlockSpec(memory_space=pltpu.VMEM),
        scratch_shapes=[
            pltpu.VMEM((m, n), jnp.bfloat16),
            pltpu.VMEM((m, n), jnp.bfloat16),
            pltpu.VMEM((m, n), jnp.bfloat16),
            pltpu.VMEM((1, m), jnp.int32),
            pltpu.VMEM((1, m), jnp.int32),
            pltpu.SemaphoreType.DMA((N_CH + 1,)),
            pltpu.SemaphoreType.DMA((N_CH + 1,)),
        ],
        compiler_params=pltpu.CompilerParams(collective_id=0),
    )(x, dest)


# device time: 9252 ns/iter; 1.0391x vs baseline; 1.0010x over previous
import jax
import jax.numpy as jnp
from jax import lax
from jax.experimental import pallas as pl
from jax.experimental.pallas import tpu as pltpu

N_CH = 8
CH = 64


def kernel(x, dest):
    m, n = x.shape
    assert N_CH * CH == m

    def body(x_ref, dest_ref, out_ref, sbuf_ref, pbuf_ref,
             d2_ref, pd_ref, send_sems, recv_sems):
        my_x = lax.axis_index("x")
        my_y = lax.axis_index("y")
        peer = (1 - my_x, my_y)

        pbuf_ref[...] = jnp.zeros((m, n), jnp.bfloat16)

        barrier_sem = pltpu.get_barrier_semaphore()
        pl.semaphore_signal(
            barrier_sem, inc=1, device_id=peer,
            device_id_type=pl.DeviceIdType.MESH,
        )
        pl.semaphore_wait(barrier_sem, 1)

        d2_ref[...] = dest_ref[...].reshape(1, m)
        rd = pltpu.make_async_remote_copy(
            src_ref=d2_ref,
            dst_ref=pd_ref,
            send_sem=send_sems.at[N_CH],
            recv_sem=recv_sems.at[N_CH],
            device_id=peer,
            device_id_type=pl.DeviceIdType.MESH,
        )
        rd.start()

        d2 = d2_ref[...]
        keep = d2 == my_x
        kf = jnp.where(keep, 1.0, 0.0)
        ksum = kf
        sh = 1
        while sh < m:
            ksum = ksum + jnp.concatenate(
                [jnp.zeros((1, sh), jnp.float32), ksum[:, : m - sh]], axis=1
            )
            sh *= 2
        ksum_i = ksum.astype(jnp.int32)
        lane = lax.broadcasted_iota(jnp.int32, (1, m), 1)
        srank = lane - ksum_i
        c_keep = jnp.sum(kf).astype(jnp.int32)
        c_send = m - c_keep

        spos = jnp.where(keep, -1, srank)
        jt = lax.broadcasted_iota(jnp.int32, (m, m), 0)
        p_send = jnp.where(jt == spos, 1.0, 0.0)
        sbuf_ref[...] = jnp.dot(
            p_send, x_ref[...], preferred_element_type=jnp.float32
        ).astype(jnp.bfloat16)

        n_send = (c_send + CH - 1) // CH
        descs = []
        for i in range(N_CH):
            desc = pltpu.make_async_remote_copy(
                src_ref=sbuf_ref.at[pl.ds(i * CH, CH), :],
                dst_ref=pbuf_ref.at[pl.ds(i * CH, CH), :],
                send_sem=send_sems.at[i],
                recv_sem=recv_sems.at[i],
                device_id=peer,
                device_id_type=pl.DeviceIdType.MESH,
            )
            descs.append(desc)

            @pl.when(i < n_send)
            def _(desc=desc):
                desc.start()

        rd.wait()
        pd = pd_ref[...]
        c_recv = jnp.sum(jnp.where(pd == my_x, 1, 0)).astype(jnp.int32)
        own_off = jnp.where(my_x == 0, 0, c_recv)
        peer_off = jnp.where(my_x == 0, c_keep, 0)

        own_pos = jnp.where(keep, ksum_i - 1 + own_off, -1)
        p_own = jnp.where(jt == own_pos, 1.0, 0.0)
        acc = jnp.dot(p_own, x_ref[...], preferred_element_type=jnp.float32)

        tt = lax.broadcasted_iota(jnp.int32, (m, m), 1)
        p_peer = jnp.where(
            (jt - tt == peer_off) & (tt < c_recv), 1.0, 0.0
        ).astype(jnp.bfloat16)

        n_recv = (c_recv + CH - 1) // CH
        for i in range(N_CH):
            @pl.when(i < n_recv)
            def _(desc=descs[i]):
                desc.wait_recv()

        out_ref[...] = acc + jnp.dot(
            p_peer, pbuf_ref[...], preferred_element_type=jnp.float32
        )

        for i in range(N_CH):
            @pl.when(i < n_send)
            def _(desc=descs[i]):
                desc.wait_send()

    return pl.pallas_call(
        body,
        out_shape=jax.ShapeDtypeStruct((m, n), jnp.float32),
        in_specs=[
            pl.BlockSpec(memory_space=pltpu.VMEM),
            pl.BlockSpec(memory_space=pltpu.VMEM),
        ],
        out_specs=pl.BlockSpec(memory_space=pltpu.VMEM),
        scratch_shapes=[
            pltpu.VMEM((m, n), jnp.bfloat16),
            pltpu.VMEM((m, n), jnp.bfloat16),
            pltpu.VMEM((1, m), jnp.int32),
            pltpu.VMEM((1, m), jnp.int32),
            pltpu.SemaphoreType.DMA((N_CH + 1,)),
            pltpu.SemaphoreType.DMA((N_CH + 1,)),
        ],
        compiler_params=pltpu.CompilerParams(collective_id=0),
    )(x, dest)
